# Initial kernel scaffold; baseline (speedup 1.0000x reference)
import jax
import jax.numpy as jnp
from jax.experimental import pallas as pl


def kernel(x_news, edge_index, W_emb, b_emb, Wl1, Wr1, att1, bg1, Wl2, Wr2, att2, bg2, Wq, bq, Wk, bk, Wv, bv, Wo, bo, ln_g, ln_b, W1, b1, W2, b2):
    raise NotImplementedError("write your pallas kernel here")


# speedup vs baseline: 38.0882x; 38.0882x over previous
import dataclasses
import functools

import jax
import jax.numpy as jnp
from jax import lax
from jax.experimental import pallas as pl
from jax.experimental.pallas import tpu as pltpu
from jax.experimental.pallas import tpu_sc as plsc

N = 10000
E = 320000
D_IN = 128
H = 64
HEADS = 4
DH = 16

NT = 10240
BB = 112
NW = 32
CH = 93
EP = NW * BB * CH
TW = 128
RPT = NT // 16
_ZC = [(i * BB, BB) for i in range(RPT // BB)] + (
    [(RPT - RPT % BB, RPT % BB)] if RPT % BB else [])


def _tc_pre(x_news_p, W_emb, b_emb, Wl, Wr):
    def body(xn, we, be, wl, wr, xc_o):
        x = jnp.dot(xn[...], we[...], preferred_element_type=jnp.float32) + be[...]
        xc_o[:, :H] = jnp.dot(x, wl[...], preferred_element_type=jnp.float32)
        xc_o[:, H:] = jnp.dot(x, wr[...], preferred_element_type=jnp.float32)

    return pl.pallas_call(
        body,
        out_shape=jax.ShapeDtypeStruct((NT, TW), jnp.float32),
    )(x_news_p, W_emb, b_emb.reshape(1, H), Wl, Wr)


def _combine(a0, a1, bg):
    num = a0[:, :H] + a1[:, :H]
    den = a0[:, H:] + a1[:, H:]
    return jnp.maximum(num / (den + 1e-16) + bg, 0.0)


def _tc_mid(acc, bg, Wl, Wr):
    def body(a, b, wl, wr, xc_o):
        x1 = _combine(a[0], a[1], b[...])
        xc_o[:, :H] = jnp.dot(x1, wl[...], preferred_element_type=jnp.float32)
        xc_o[:, H:] = jnp.dot(x1, wr[...], preferred_element_type=jnp.float32)

    return pl.pallas_call(
        body,
        out_shape=jax.ShapeDtypeStruct((NT, TW), jnp.float32),
    )(acc, bg.reshape(1, H), Wl, Wr)


def _tc_post(acc, bg, Wv, bv, Wo, bo, ln_g, ln_b, W1, b1, W2, b2):
    def body(a, bgr, wv, bvr, wo, bor, lg, lb, w1, b1r, w2, b2r, out):
        x2 = _combine(a[0], a[1], bgr[...])
        v = jnp.dot(x2, wv[...], preferred_element_type=jnp.float32) + bvr[...]
        att_o = jnp.dot(v, wo[...], preferred_element_type=jnp.float32) + bor[...]
        y = x2 + att_o
        mu = jnp.mean(y, axis=1, keepdims=True)
        var = jnp.mean((y - mu) ** 2, axis=1, keepdims=True)
        yn = (y - mu) * lax.rsqrt(var + 1e-5) * lg[...] + lb[...]
        h1 = jnp.maximum(jnp.dot(yn, w1[...], preferred_element_type=jnp.float32)
                         + b1r[...], 0.0)
        out[...] = jnp.dot(h1, w2[...], preferred_element_type=jnp.float32) + b2r[...]

    return pl.pallas_call(
        body,
        out_shape=jax.ShapeDtypeStruct((NT, 2), jnp.float32),
    )(acc, bg.reshape(1, H), Wv, bv.reshape(1, H), Wo, bo.reshape(1, H),
      ln_g.reshape(1, H), ln_b.reshape(1, H), W1, b1.reshape(1, H // 2),
      W2, b2.reshape(1, 2))


def _edge_pass(xc, src_p, dst_p, att_p):
    mesh = plsc.VectorSubcoreMesh(core_axis_name="c", subcore_axis_name="s")
    cp = pltpu.CompilerParams()
    if "needs_layout_passes" in pltpu.CompilerParams.__dataclass_fields__:
        cp = dataclasses.replace(cp, needs_layout_passes=False)

    @functools.partial(
        pl.kernel,
        mesh=mesh,
        compiler_params=cp,
        out_type=jax.ShapeDtypeStruct((2, NT, TW), jnp.float32),
        scratch_types=[
            pltpu.VMEM((1, BB), jnp.int32),
            pltpu.VMEM((1, BB), jnp.int32),
            pltpu.VMEM((BB, TW), jnp.float32),
            pltpu.VMEM((BB, TW), jnp.float32),
            pltpu.VMEM((BB, TW), jnp.float32),
            pltpu.VMEM((8, 128), jnp.float32),
            pltpu.VMEM_SHARED((NT, TW), jnp.float32),
        ],
    )
    def k(xc_h, src_h, dst_h, att_h, out_h, si, di, xsr, xdr, msg, attv, acc):
        c = lax.axis_index("c")
        s = lax.axis_index("s")
        wid = s * 2 + c

        pltpu.sync_copy(att_h, attv)

        @pl.loop(0, BB)
        def _(e):
            for k5 in range(TW // DH):
                msg[e, pl.ds(k5 * DH, DH)] = jnp.zeros((DH,), jnp.float32)

        rows0 = s * RPT
        for off, nrow in _ZC:
            pltpu.sync_copy(msg.at[pl.ds(0, nrow)],
                            acc.at[pl.ds(rows0 + off, nrow)])
        plsc.subcore_barrier()

        att_regs = [attv[h, pl.ds(0, DH)] for h in range(HEADS)]

        @pl.loop(0, CH)
        def _(j):
            base = (wid * CH + j) * BB
            pltpu.sync_copy(src_h.at[pl.ds(base, BB)], si.at[0])
            pltpu.sync_copy(dst_h.at[pl.ds(base, BB)], di.at[0])
            pltpu.sync_copy(xc_h.at[si.at[0]], xsr)
            pltpu.sync_copy(xc_h.at[di.at[0]], xdr)

            @pl.loop(0, BB)
            def _(e):
                for h in range(HEADS):
                    vl = xsr[e, pl.ds(h * DH, DH)]
                    vr = xdr[e, pl.ds(H + h * DH, DH)]
                    sv = vl + vr
                    lk = jnp.maximum(sv, sv * 0.2)
                    a = jnp.sum(lk * att_regs[h])
                    ex = jnp.exp(jnp.full((DH,), a, jnp.float32))
                    msg[e, pl.ds(h * DH, DH)] = vl * ex
                    msg[e, pl.ds(H + h * DH, DH)] = ex

            pltpu.sync_copy(msg, acc.at[di.at[0]], add=True)

        plsc.subcore_barrier()
        for off, nrow in _ZC:
            r0 = rows0 + off
            pltpu.sync_copy(acc.at[pl.ds(r0, nrow)],
                            out_h.at[c, pl.ds(r0, nrow)])

    return k(xc, src_p, dst_p, att_p)


def kernel(x_news, edge_index, W_emb, b_emb, Wl1, Wr1, att1, bg1, Wl2, Wr2,
           att2, bg2, Wq, bq, Wk, bk, Wv, bv, Wo, bo, ln_g, ln_b, W1, b1,
           W2, b2):
    xp = jnp.zeros((NT, D_IN), jnp.float32).at[:N].set(x_news)
    loop_idx = jnp.arange(N, dtype=jnp.int32)
    pad_idx = jnp.full((EP - E - N,), NT - 1, jnp.int32)
    src = jnp.concatenate([edge_index[0], loop_idx, pad_idx])
    dst = jnp.concatenate([edge_index[1], loop_idx, pad_idx])
    att1_p = jnp.zeros((8, 128), jnp.float32).at[:HEADS, :DH].set(att1)
    att2_p = jnp.zeros((8, 128), jnp.float32).at[:HEADS, :DH].set(att2)

    xc1 = _tc_pre(xp, W_emb, b_emb, Wl1, Wr1)
    acc1 = _edge_pass(xc1, src, dst, att1_p)
    xc2 = _tc_mid(acc1, bg1, Wl2, Wr2)
    acc2 = _edge_pass(xc2, src, dst, att2_p)
    logits = _tc_post(acc2, bg2, Wv, bv, Wo, bo, ln_g, ln_b, W1, b1, W2, b2)
    return logits[:N]


# speedup vs baseline: 66.2915x; 1.7405x over previous
import dataclasses
import functools

import jax
import jax.numpy as jnp
from jax import lax
from jax.experimental import pallas as pl
from jax.experimental.pallas import tpu as pltpu
from jax.experimental.pallas import tpu_sc as plsc

N = 10000
E = 320000
D_IN = 128
H = 64
HEADS = 4
DH = 16

NT = 10112
BB = 64
NW = 32
CH = 162
EP = NW * BB * CH
TW = 128
RPT = NT // 16
_ZC = [(i * BB, BB) for i in range(RPT // BB)] + (
    [(RPT - RPT % BB, RPT % BB)] if RPT % BB else [])


def _tc_pre(x_news_p, W_emb, b_emb, Wl, Wr):
    def body(xn, we, be, wl, wr, xc_o):
        x = jnp.dot(xn[...], we[...], preferred_element_type=jnp.float32) + be[...]
        xc_o[:, :H] = jnp.dot(x, wl[...], preferred_element_type=jnp.float32)
        xc_o[:, H:] = jnp.dot(x, wr[...], preferred_element_type=jnp.float32)

    return pl.pallas_call(
        body,
        out_shape=jax.ShapeDtypeStruct((NT, TW), jnp.float32),
    )(x_news_p, W_emb, b_emb.reshape(1, H), Wl, Wr)


def _combine(a0, a1, bg):
    num = a0[:, :H] + a1[:, :H]
    den = a0[:, H:] + a1[:, H:]
    return jnp.maximum(num / (den + 1e-16) + bg, 0.0)


def _tc_mid(acc, bg, Wl, Wr):
    def body(a, b, wl, wr, xc_o):
        x1 = _combine(a[0], a[1], b[...])
        xc_o[:, :H] = jnp.dot(x1, wl[...], preferred_element_type=jnp.float32)
        xc_o[:, H:] = jnp.dot(x1, wr[...], preferred_element_type=jnp.float32)

    return pl.pallas_call(
        body,
        out_shape=jax.ShapeDtypeStruct((NT, TW), jnp.float32),
    )(acc, bg.reshape(1, H), Wl, Wr)


def _tc_post(acc, bg, Wv, bv, Wo, bo, ln_g, ln_b, W1, b1, W2, b2):
    def body(a, bgr, wv, bvr, wo, bor, lg, lb, w1, b1r, w2, b2r, out):
        x2 = _combine(a[0], a[1], bgr[...])
        v = jnp.dot(x2, wv[...], preferred_element_type=jnp.float32) + bvr[...]
        att_o = jnp.dot(v, wo[...], preferred_element_type=jnp.float32) + bor[...]
        y = x2 + att_o
        mu = jnp.mean(y, axis=1, keepdims=True)
        var = jnp.mean((y - mu) ** 2, axis=1, keepdims=True)
        yn = (y - mu) * lax.rsqrt(var + 1e-5) * lg[...] + lb[...]
        h1 = jnp.maximum(jnp.dot(yn, w1[...], preferred_element_type=jnp.float32)
                         + b1r[...], 0.0)
        out[...] = jnp.dot(h1, w2[...], preferred_element_type=jnp.float32) + b2r[...]

    return pl.pallas_call(
        body,
        out_shape=jax.ShapeDtypeStruct((NT, 2), jnp.float32),
    )(acc, bg.reshape(1, H), Wv, bv.reshape(1, H), Wo, bo.reshape(1, H),
      ln_g.reshape(1, H), ln_b.reshape(1, H), W1, b1.reshape(1, H // 2),
      W2, b2.reshape(1, 2))


def _edge_pass(xc, src_p, dst_p, att_p):
    mesh = plsc.VectorSubcoreMesh(core_axis_name="c", subcore_axis_name="s")
    cp = pltpu.CompilerParams()
    if "needs_layout_passes" in pltpu.CompilerParams.__dataclass_fields__:
        cp = dataclasses.replace(cp, needs_layout_passes=False)

    @functools.partial(
        pl.kernel,
        mesh=mesh,
        compiler_params=cp,
        out_type=jax.ShapeDtypeStruct((2, NT, TW), jnp.float32),
        scratch_types=[
            pltpu.VMEM((2, 1, BB), jnp.int32),
            pltpu.VMEM((2, 1, BB), jnp.int32),
            pltpu.VMEM((BB, TW), jnp.float32),
            pltpu.VMEM((2, BB, TW), jnp.float32),
            pltpu.VMEM((2, BB, TW), jnp.float32),
            pltpu.VMEM((4, 128), jnp.float32),
            pltpu.VMEM_SHARED((NT, TW), jnp.float32),
            pltpu.SemaphoreType.DMA((2,)),
            pltpu.SemaphoreType.DMA((2,)),
        ],
    )
    def k(xc_h, src_h, dst_h, att_h, out_h, si, di, msg, xsr, xdr, attv,
          acc, sidx, sg):
        c = lax.axis_index("c")
        s = lax.axis_index("s")
        wid = s * 2 + c

        pltpu.sync_copy(att_h.at[pl.ds(0, HEADS)], attv)

        @pl.loop(0, BB)
        def _(e):
            for k5 in range(TW // DH):
                msg[e, pl.ds(k5 * DH, DH)] = jnp.zeros((DH,), jnp.float32)

        rows0 = s * RPT
        for off, nrow in _ZC:
            pltpu.sync_copy(msg.at[pl.ds(0, nrow)],
                            acc.at[pl.ds(rows0 + off, nrow)])
        plsc.subcore_barrier()

        att_regs = [attv[h, pl.ds(0, DH)] for h in range(HEADS)]
        base0 = wid * CH * BB

        def start_idx(j, p):
            pltpu.async_copy(src_h.at[pl.ds(base0 + j * BB, BB)],
                             si.at[p, 0], sidx.at[p])
            pltpu.async_copy(dst_h.at[pl.ds(base0 + j * BB, BB)],
                             di.at[p, 0], sidx.at[p])

        def wait_idx(j, p):
            pltpu.make_async_copy(src_h.at[pl.ds(base0 + j * BB, BB)],
                                  si.at[p, 0], sidx.at[p]).wait()
            pltpu.make_async_copy(dst_h.at[pl.ds(base0 + j * BB, BB)],
                                  di.at[p, 0], sidx.at[p]).wait()

        def start_gather(p):
            pltpu.async_copy(xc_h.at[si.at[p, 0]], xsr.at[p], sg.at[p])
            pltpu.async_copy(xc_h.at[di.at[p, 0]], xdr.at[p], sg.at[p])

        def wait_gather(p):
            pltpu.make_async_copy(xc_h.at[si.at[p, 0]], xsr.at[p],
                                  sg.at[p]).wait()
            pltpu.make_async_copy(xc_h.at[di.at[p, 0]], xdr.at[p],
                                  sg.at[p]).wait()

        start_idx(0, 0)
        start_idx(1, 1)
        wait_idx(0, 0)
        start_gather(0)

        def chunk(j, p, q):
            wait_gather(p)

            @pl.when(j + 1 < CH)
            def _():
                wait_idx(j + 1, q)
                start_gather(q)

            @pl.loop(0, BB)
            def _(e):
                for h in range(HEADS):
                    vl = xsr[p, e, pl.ds(h * DH, DH)]
                    vr = xdr[p, e, pl.ds(H + h * DH, DH)]
                    sv = vl + vr
                    lk = jnp.maximum(sv, sv * 0.2)
                    a = jnp.sum(lk * att_regs[h])
                    ex = jnp.exp(jnp.full((DH,), a, jnp.float32))
                    msg[e, pl.ds(h * DH, DH)] = vl * ex
                    msg[e, pl.ds(H + h * DH, DH)] = ex

            pltpu.sync_copy(msg, acc.at[di.at[p, 0]], add=True)

            @pl.when(j + 2 < CH)
            def _():
                start_idx(j + 2, p)

        @pl.loop(0, CH // 2)
        def _(jj):
            chunk(2 * jj, 0, 1)
            chunk(2 * jj + 1, 1, 0)

        plsc.subcore_barrier()
        for off, nrow in _ZC:
            r0 = rows0 + off
            pltpu.sync_copy(acc.at[pl.ds(r0, nrow)],
                            out_h.at[c, pl.ds(r0, nrow)])

    return k(xc, src_p, dst_p, att_p)


def kernel(x_news, edge_index, W_emb, b_emb, Wl1, Wr1, att1, bg1, Wl2, Wr2,
           att2, bg2, Wq, bq, Wk, bk, Wv, bv, Wo, bo, ln_g, ln_b, W1, b1,
           W2, b2):
    xp = jnp.zeros((NT, D_IN), jnp.float32).at[:N].set(x_news)
    loop_idx = jnp.arange(N, dtype=jnp.int32)
    pad_idx = jnp.full((EP - E - N,), NT - 1, jnp.int32)
    src = jnp.concatenate([edge_index[0], loop_idx, pad_idx])
    dst = jnp.concatenate([edge_index[1], loop_idx, pad_idx])
    att1_p = jnp.zeros((8, 128), jnp.float32).at[:HEADS, :DH].set(att1)
    att2_p = jnp.zeros((8, 128), jnp.float32).at[:HEADS, :DH].set(att2)

    xc1 = _tc_pre(xp, W_emb, b_emb, Wl1, Wr1)
    acc1 = _edge_pass(xc1, src, dst, att1_p)
    xc2 = _tc_mid(acc1, bg1, Wl2, Wr2)
    acc2 = _edge_pass(xc2, src, dst, att2_p)
    logits = _tc_post(acc2, bg2, Wv, bv, Wo, bo, ln_g, ln_b, W1, b1, W2, b2)
    return logits[:N]
